# baseline (speedup 1.0000x reference)
import functools

import jax
import jax.numpy as jnp
from jax import lax
from jax.experimental import pallas as pl
from jax.experimental.pallas import tpu as pltpu
from jax.experimental.pallas import tpu_sc as plsc

N_USERS = 25000
N_ITEMS = 25000
N_NODES = N_USERS + N_ITEMS
DIM = 64
HALF = DIM // 2
N_EDGES = 800000
CHUNK = 128
NC = 2
NS = 16
LANES = 16
SUPER = 3
SUP_E = SUPER * CHUNK
N_CHUNKS = N_EDGES // CHUNK
SUPS_PT = 130

NPAD = 50048
ROWS_PT = NPAD // NS
DEG_PAD = NPAD
DEG_PT = ROWS_PT

_mesh = plsc.VectorSubcoreMesh(core_axis_name="c", subcore_axis_name="s")


DEG_SUPS = 64


@functools.partial(
    pl.kernel,
    mesh=_mesh,
    out_type=jax.ShapeDtypeStruct((NC * DEG_PAD,), jnp.float32),
    scratch_types=(
        [pltpu.VMEM((CHUNK,), jnp.int32)] * 8
        + [
            pltpu.VMEM((CHUNK,), jnp.float32),
            pltpu.VMEM((SUP_E,), jnp.float32),
            pltpu.VMEM((DEG_PT,), jnp.float32),
            pltpu.VMEM_SHARED((DEG_PAD,), jnp.float32),
            pltpu.SemaphoreType.DMA,
            pltpu.SemaphoreType.DMA,
        ]
    ),
    compiler_params=pltpu.CompilerParams(use_tc_tiling_on_sc=False),
)
def _sc_degree(col_hbm, deg_hbm, k00, k01, k02, k10, k11, k12, kt0, kt1,
               ones, zeros, stage, deg_sh, semi0, semi1):
    c = lax.axis_index("c")
    s = lax.axis_index("s")

    one16 = jnp.ones((LANES,), jnp.float32)
    zero16 = jnp.zeros((LANES,), jnp.float32)
    for j in range(CHUNK // LANES):
        ones[pl.ds(j * LANES, LANES)] = one16
    for j in range(SUP_E // LANES):
        zeros[pl.ds(j * LANES, LANES)] = zero16

    stripe0 = s * DEG_PT

    def _zinit(k, _):
        pltpu.sync_copy(zeros, deg_sh.at[pl.ds(stripe0 + k * SUP_E, SUP_E)])
        return _

    lax.fori_loop(0, DEG_PT // SUP_E, _zinit, None)
    pltpu.sync_copy(zeros.at[pl.ds(0, DEG_PT % SUP_E)],
                    deg_sh.at[pl.ds(stripe0 + (DEG_PT // SUP_E) * SUP_E,
                                    DEG_PT % SUP_E)])
    plsc.subcore_barrier()

    half_chunks = N_CHUNKS // NC
    base = half_chunks // NS
    start = c * half_chunks + s * base + jnp.minimum(s, 5)

    cidx = ((k00, k01, k02), (k10, k11, k12))
    semi = (semi0, semi1)

    def fire(b, sup):
        e0 = (start + sup * SUPER) * CHUNK
        for j in range(SUPER):
            pltpu.async_copy(col_hbm.at[pl.ds(e0 + j * CHUNK, CHUNK)],
                             cidx[b][j], semi[b])

    def drain_scat(b):
        for j in range(SUPER):
            pltpu.make_async_copy(col_hbm.at[pl.ds(0, CHUNK)], cidx[b][j],
                                  semi[b]).wait()
        for j in range(SUPER):
            pltpu.sync_copy(ones, deg_sh.at[cidx[b][j]], add=True)

    fire(0, 0)

    def _body(m, _):
        fire(1, 2 * m + 1)
        drain_scat(0)
        fire(0, 2 * m + 2)
        drain_scat(1)
        return _

    lax.fori_loop(0, (DEG_SUPS - 2) // 2, _body, None)
    fire(1, DEG_SUPS - 1)
    drain_scat(0)
    drain_scat(1)

    def tail_one(t, kref):
        e0 = (start + DEG_SUPS * SUPER + t) * CHUNK
        pltpu.sync_copy(col_hbm.at[pl.ds(e0, CHUNK)], kref)
        pltpu.sync_copy(ones, deg_sh.at[kref], add=True)

    tail_one(0, kt0)
    tail_one(1, kt1)
    tail_one(2, k00)
    pl.when(s < 5)(lambda: tail_one(3, k10))
    plsc.subcore_barrier()

    pltpu.sync_copy(deg_sh.at[pl.ds(stripe0, DEG_PT)], stage)
    pltpu.sync_copy(stage, deg_hbm.at[pl.ds(c * DEG_PAD + stripe0, DEG_PT)])


def _newton_factor(d, mode):
    i = plsc.bitcast(d, jnp.int32)
    if mode == "recip":
        r = plsc.bitcast(jnp.int32(0x7EF311C3) - i, jnp.float32)
        for _ in range(3):
            r = r * (2.0 - d * r)
    else:
        r = plsc.bitcast(jnp.int32(0x5F3759DF) - (i >> 1), jnp.float32)
        for _ in range(3):
            r = r * (1.5 - 0.5 * d * r * r)
    return jnp.where(d > 0.0, r, 0.0)


def _make_propagate(mode, prescale):
    n_out = 4 if prescale else 2

    @functools.partial(
        pl.kernel,
        mesh=_mesh,
        out_type=[jax.ShapeDtypeStruct((NPAD, HALF), jnp.float32)] * n_out,
        scratch_types=(
            [pltpu.VMEM((SUP_E,), jnp.int32)] * 2
            + [pltpu.VMEM((CHUNK,), jnp.int32)] * 6
            + [pltpu.VMEM((SUP_E, HALF), jnp.float32)] * 2
            + [pltpu.VMEM((SUP_E + LANES,), jnp.float32)] * 2
            + [
                pltpu.VMEM_SHARED((NPAD, HALF), jnp.float32),
                pltpu.SemaphoreType.DMA,
                pltpu.SemaphoreType.DMA,
                pltpu.SemaphoreType.DMA,
                pltpu.SemaphoreType.DMA,
                pltpu.SemaphoreType.DMA,
                pltpu.SemaphoreType.DMA,
            ]
        ),
        compiler_params=pltpu.CompilerParams(use_tc_tiling_on_sc=False,
                                             needs_layout_passes=False),
    )
    def _sc_propagate(ylo_hbm, yhi_hbm, row_hbm, col_hbm, deg_hbm,
                      zlo_hbm, zhi_hbm, *rest):
        if prescale:
            (yslo_hbm, yshi_hbm, ridx0, ridx1,
             c00, c01, c02, c10, c11, c12,
             rows0, rows1, fa, fb, z_sh,
             semg0, semg1, semi0, semi1, sems0, sems1) = rest
            gsrc = (yslo_hbm, yshi_hbm)
        else:
            (ridx0, ridx1,
             c00, c01, c02, c10, c11, c12,
             rows0, rows1, fa, fb, z_sh,
             semg0, semg1, semi0, semi1, sems0, sems1) = rest
            gsrc = (ylo_hbm, yhi_hbm)
        c = lax.axis_index("c")
        s = lax.axis_index("s")

        zero16 = jnp.zeros((LANES,), jnp.float32)

        def _zb(i, _):
            for j in range(HALF // LANES):
                rows0[i, pl.ds(j * LANES, LANES)] = zero16
            return _

        lax.fori_loop(0, SUP_E, _zb, None)

        stripe0 = s * ROWS_PT

        def _zinit(k, _):
            pltpu.sync_copy(rows0, z_sh.at[pl.ds(stripe0 + k * SUP_E, SUP_E)])
            return _

        lax.fori_loop(0, ROWS_PT // SUP_E, _zinit, None)
        pltpu.sync_copy(rows0.at[pl.ds(0, ROWS_PT % SUP_E)],
                        z_sh.at[pl.ds(stripe0 + (ROWS_PT // SUP_E) * SUP_E,
                                      ROWS_PT % SUP_E)])

        def _scale_rows(buf, nrows, m):
            nv = (nrows + LANES - 1) // LANES
            for j in range(nv):
                sl = pl.ds(j * LANES, LANES)
                d = fa[sl] + fb[sl]
                fa[sl] = _newton_factor(d, m)

            def _sr(i, _):
                v = fa[pl.ds(i, LANES)]
                f16 = jnp.full((LANES,), v[0], jnp.float32)
                for j in range(HALF // LANES):
                    sl = pl.ds(j * LANES, LANES)
                    buf[i, sl] = buf[i, sl] * f16
                return _

            lax.fori_loop(0, nrows, _sr, None)

        def _load_deg(r0, n):
            pltpu.sync_copy(deg_hbm.at[pl.ds(r0, n)], fa.at[pl.ds(0, n)])
            pltpu.sync_copy(deg_hbm.at[pl.ds(DEG_PAD + r0, n)],
                            fb.at[pl.ds(0, n)])

        if prescale:
            def _pre(x_ref, ydst):
                def _pc(k, _):
                    r0 = stripe0 + k * SUP_E
                    pltpu.sync_copy(x_ref.at[pl.ds(r0, SUP_E)], rows1)
                    _load_deg(r0, SUP_E)
                    _scale_rows(rows1, SUP_E, "rsqrt")
                    pltpu.sync_copy(rows1, ydst.at[pl.ds(r0, SUP_E)])
                    return _

                lax.fori_loop(0, ROWS_PT // SUP_E, _pc, None)

                def _ptail(tsz):
                    def f():
                        t0 = stripe0 + (ROWS_PT // SUP_E) * SUP_E
                        pltpu.sync_copy(x_ref.at[pl.ds(t0, tsz)],
                                        rows1.at[pl.ds(0, tsz)])
                        _load_deg(t0, tsz)
                        _scale_rows(rows1, tsz, "rsqrt")
                        pltpu.sync_copy(rows1.at[pl.ds(0, tsz)],
                                        ydst.at[pl.ds(t0, tsz)])
                    return f

                pl.when(s < NS - 1)(_ptail(ROWS_PT % SUP_E))
                pl.when(s == NS - 1)(_ptail(N_NODES - (NS - 1) * ROWS_PT
                                            - (ROWS_PT // SUP_E) * SUP_E))

            pl.when(c == 0)(lambda: _pre(ylo_hbm, yslo_hbm))
            pl.when(c == 1)(lambda: _pre(yhi_hbm, yshi_hbm))

        plsc.subcore_barrier()

        ridx = (ridx0, ridx1)
        cidx = ((c00, c01, c02), (c10, c11, c12))
        rows = (rows0, rows1)
        semg = (semg0, semg1)
        semi = (semi0, semi1)
        sems = (sems0, sems1)
        start_s = s * 390 + jnp.minimum(s, 10)

        def _run(y_ref):
            def fire(b, sup, drain_scatters):
                if drain_scatters:
                    pltpu.make_async_copy(y_ref.at[pl.ds(0, SUP_E)],
                                          rows[b], sems[b]).wait()
                e0 = (start_s + sup * SUPER) * CHUNK
                pltpu.async_copy(row_hbm.at[pl.ds(e0, SUP_E)], ridx[b],
                                 semi[b])
                for j in range(SUPER):
                    pltpu.async_copy(col_hbm.at[pl.ds(e0 + j * CHUNK, CHUNK)],
                                     cidx[b][j], semi[b])
                pltpu.make_async_copy(y_ref.at[pl.ds(0, SUP_E // 16)],
                                      rows[b].at[pl.ds(0, SUP_E // 16)],
                                      semi[b]).wait()
                for j in range(SUPER):
                    sl = pl.ds(j * CHUNK, CHUNK)
                    pltpu.async_copy(y_ref.at[ridx[b].at[sl]], rows[b].at[sl],
                                     semg[b])

            def process(b):
                pltpu.make_async_copy(y_ref.at[pl.ds(0, SUP_E)], rows[b],
                                      semg[b]).wait()
                for j in range(SUPER):
                    pltpu.async_copy(rows[b].at[pl.ds(j * CHUNK, CHUNK)],
                                     z_sh.at[cidx[b][j]], sems[b], add=True)

            fire(0, 0, False)
            fire(1, 1, False)
            process(0)

            def _body(m, _):
                fire(0, 2 * m + 2, True)
                process(1)
                fire(1, 2 * m + 3, True)
                process(0)
                return _

            lax.fori_loop(0, (SUPS_PT - 2) // 2, _body, None)
            process(1)
            pltpu.make_async_copy(y_ref.at[pl.ds(0, SUP_E)], rows[0],
                                  sems[0]).wait()
            pltpu.make_async_copy(y_ref.at[pl.ds(0, SUP_E)], rows[1],
                                  sems[1]).wait()

            def tail_one():
                e0 = (start_s + SUPS_PT * SUPER) * CHUNK
                sl = pl.ds(0, CHUNK)
                pltpu.sync_copy(row_hbm.at[pl.ds(e0, CHUNK)], ridx[0].at[sl])
                pltpu.sync_copy(col_hbm.at[pl.ds(e0, CHUNK)], cidx[0][0])
                pltpu.async_copy(y_ref.at[ridx[0].at[sl]], rows[0].at[sl],
                                 semg[0]).wait()
                pltpu.sync_copy(rows[0].at[sl], z_sh.at[cidx[0][0]],
                                add=True)

            pl.when(s < 10)(tail_one)

        pl.when(c == 0)(lambda: _run(gsrc[0]))
        pl.when(c == 1)(lambda: _run(gsrc[1]))
        plsc.subcore_barrier()

        def _cpout_to(dst):
            def _cp(k, _):
                r0 = stripe0 + k * SUP_E
                pltpu.sync_copy(z_sh.at[pl.ds(r0, SUP_E)], rows0)
                _load_deg(r0, SUP_E)
                _scale_rows(rows0, SUP_E, mode)
                pltpu.sync_copy(rows0, dst.at[pl.ds(r0, SUP_E)])
                return _

            lax.fori_loop(0, ROWS_PT // SUP_E, _cp, None)
            tail = ROWS_PT % SUP_E
            t0 = stripe0 + (ROWS_PT // SUP_E) * SUP_E
            pltpu.sync_copy(z_sh.at[pl.ds(t0, tail)], rows0.at[pl.ds(0, tail)])
            _load_deg(t0, tail)
            _scale_rows(rows0, tail, mode)
            pltpu.sync_copy(rows0.at[pl.ds(0, tail)], dst.at[pl.ds(t0, tail)])

        pl.when(c == 0)(lambda: _cpout_to(zlo_hbm))
        pl.when(c == 1)(lambda: _cpout_to(zhi_hbm))

    return _sc_propagate


_sc_propagate_l1 = _make_propagate("recip", True)
_sc_propagate_l2 = _make_propagate("rsqrt", False)


def _tc_dis_kernel(d0_ref, d1_ref, sqd_ref):
    deg = d0_ref[...] + d1_ref[...]
    sqd_ref[...] = jnp.where(deg > 0.0, jnp.sqrt(deg), 0.0)


def _tc_final_kernel(x_ref, sqd_ref, w1lo_ref, w1hi_ref, x2lo_ref, x2hi_ref,
                     out_ref):
    sq = sqd_ref[...]
    w1 = jnp.concatenate([w1lo_ref[...], w1hi_ref[...]], axis=1)
    x2 = jnp.concatenate([x2lo_ref[...], x2hi_ref[...]], axis=1)
    out_ref[...] = (x_ref[...] + sq * w1 + x2) * (1.0 / 3.0)


_R = 1000
_GRID = N_NODES // _R


def _row_spec(w):
    return pl.BlockSpec((_R, w), lambda i: (i, 0))


def kernel(user_weight, item_weight, edge_index):
    x0 = jnp.concatenate([user_weight, item_weight], axis=0)
    rowv = edge_index[0]
    colv = edge_index[1]

    xlo = jnp.concatenate([user_weight[:, :HALF], item_weight[:, :HALF]],
                          axis=0)
    xhi = jnp.concatenate([user_weight[:, HALF:], item_weight[:, HALF:]],
                          axis=0)

    degp = _sc_degree(colv)
    d0 = degp[:N_NODES].reshape(50, 1000)
    d1 = degp[DEG_PAD:DEG_PAD + N_NODES].reshape(50, 1000)

    sqd2 = pl.pallas_call(
        _tc_dis_kernel,
        out_shape=jax.ShapeDtypeStruct((50, 1000), jnp.float32),
    )(d0, d1)
    sqd = sqd2.reshape(N_NODES, 1)

    w1lo, w1hi, _, _ = _sc_propagate_l1(xlo, xhi, rowv, colv, degp)
    x2lo, x2hi = _sc_propagate_l2(w1lo, w1hi, rowv, colv, degp)

    out = pl.pallas_call(
        _tc_final_kernel,
        grid=(_GRID,),
        in_specs=[_row_spec(DIM), _row_spec(1), _row_spec(HALF),
                  _row_spec(HALF), _row_spec(HALF), _row_spec(HALF)],
        out_specs=_row_spec(DIM),
        out_shape=jax.ShapeDtypeStruct((N_NODES, DIM), jnp.float32),
    )(x0, sqd, w1lo, w1hi, x2lo, x2hi)

    return out[:N_USERS], out[N_USERS:]


# speedup vs baseline: 1.0204x; 1.0204x over previous
import functools

import jax
import jax.numpy as jnp
from jax import lax
from jax.experimental import pallas as pl
from jax.experimental.pallas import tpu as pltpu
from jax.experimental.pallas import tpu_sc as plsc

N_USERS = 25000
N_ITEMS = 25000
N_NODES = N_USERS + N_ITEMS
DIM = 64
HALF = DIM // 2
N_EDGES = 800000
CHUNK = 128
NC = 2
NS = 16
LANES = 16
SUPER = 3
SUP_E = SUPER * CHUNK
N_CHUNKS = N_EDGES // CHUNK
SUPS_PT = 130

NPAD = 50048
ROWS_PT = NPAD // NS
DEG_PAD = NPAD
DEG_PT = ROWS_PT

_mesh = plsc.VectorSubcoreMesh(core_axis_name="c", subcore_axis_name="s")


DEG_SUPS = 64


@functools.partial(
    pl.kernel,
    mesh=_mesh,
    out_type=jax.ShapeDtypeStruct((NC * DEG_PAD,), jnp.float32),
    scratch_types=(
        [pltpu.VMEM((CHUNK,), jnp.int32)] * 8
        + [
            pltpu.VMEM((CHUNK,), jnp.float32),
            pltpu.VMEM((SUP_E,), jnp.float32),
            pltpu.VMEM((DEG_PT,), jnp.float32),
            pltpu.VMEM_SHARED((DEG_PAD,), jnp.float32),
            pltpu.SemaphoreType.DMA,
            pltpu.SemaphoreType.DMA,
        ]
    ),
    compiler_params=pltpu.CompilerParams(use_tc_tiling_on_sc=False),
)
def _sc_degree(col_hbm, deg_hbm, k00, k01, k02, k10, k11, k12, kt0, kt1,
               ones, zeros, stage, deg_sh, semi0, semi1):
    c = lax.axis_index("c")
    s = lax.axis_index("s")

    one16 = jnp.ones((LANES,), jnp.float32)
    zero16 = jnp.zeros((LANES,), jnp.float32)
    for j in range(CHUNK // LANES):
        ones[pl.ds(j * LANES, LANES)] = one16
    for j in range(SUP_E // LANES):
        zeros[pl.ds(j * LANES, LANES)] = zero16

    stripe0 = s * DEG_PT

    def _zinit(k, _):
        pltpu.sync_copy(zeros, deg_sh.at[pl.ds(stripe0 + k * SUP_E, SUP_E)])
        return _

    lax.fori_loop(0, DEG_PT // SUP_E, _zinit, None)
    pltpu.sync_copy(zeros.at[pl.ds(0, DEG_PT % SUP_E)],
                    deg_sh.at[pl.ds(stripe0 + (DEG_PT // SUP_E) * SUP_E,
                                    DEG_PT % SUP_E)])
    plsc.subcore_barrier()

    half_chunks = N_CHUNKS // NC
    base = half_chunks // NS
    start = c * half_chunks + s * base + jnp.minimum(s, 5)

    cidx = ((k00, k01, k02), (k10, k11, k12))
    semi = (semi0, semi1)

    def fire(b, sup):
        e0 = (start + sup * SUPER) * CHUNK
        for j in range(SUPER):
            pltpu.async_copy(col_hbm.at[pl.ds(e0 + j * CHUNK, CHUNK)],
                             cidx[b][j], semi[b])

    def drain_scat(b):
        for j in range(SUPER):
            pltpu.make_async_copy(col_hbm.at[pl.ds(0, CHUNK)], cidx[b][j],
                                  semi[b]).wait()
        for j in range(SUPER):
            pltpu.sync_copy(ones, deg_sh.at[cidx[b][j]], add=True)

    fire(0, 0)

    def _body(m, _):
        fire(1, 2 * m + 1)
        drain_scat(0)
        fire(0, 2 * m + 2)
        drain_scat(1)
        return _

    lax.fori_loop(0, (DEG_SUPS - 2) // 2, _body, None)
    fire(1, DEG_SUPS - 1)
    drain_scat(0)
    drain_scat(1)

    def tail_one(t, kref):
        e0 = (start + DEG_SUPS * SUPER + t) * CHUNK
        pltpu.sync_copy(col_hbm.at[pl.ds(e0, CHUNK)], kref)
        pltpu.sync_copy(ones, deg_sh.at[kref], add=True)

    tail_one(0, kt0)
    tail_one(1, kt1)
    tail_one(2, k00)
    pl.when(s < 5)(lambda: tail_one(3, k10))
    plsc.subcore_barrier()

    pltpu.sync_copy(deg_sh.at[pl.ds(stripe0, DEG_PT)], stage)
    pltpu.sync_copy(stage, deg_hbm.at[pl.ds(c * DEG_PAD + stripe0, DEG_PT)])


def _newton_factor(d, mode):
    i = plsc.bitcast(d, jnp.int32)
    if mode == "recip":
        r = plsc.bitcast(jnp.int32(0x7EF311C3) - i, jnp.float32)
        for _ in range(3):
            r = r * (2.0 - d * r)
    else:
        r = plsc.bitcast(jnp.int32(0x5F3759DF) - (i >> 1), jnp.float32)
        for _ in range(3):
            r = r * (1.5 - 0.5 * d * r * r)
    return jnp.where(d > 0.0, r, 0.0)


def _make_propagate(mode, prescale):
    n_out = 4 if prescale else 2

    @functools.partial(
        pl.kernel,
        mesh=_mesh,
        out_type=[jax.ShapeDtypeStruct((NPAD, HALF), jnp.float32)] * n_out,
        scratch_types=(
            [pltpu.VMEM((SUP_E,), jnp.int32)] * 2
            + [pltpu.VMEM((CHUNK,), jnp.int32)] * 6
            + [pltpu.VMEM((SUP_E, HALF), jnp.float32)] * 2
            + [pltpu.VMEM((SUP_E + LANES,), jnp.float32)] * 2
            + [
                pltpu.VMEM_SHARED((NPAD, HALF), jnp.float32),
                pltpu.SemaphoreType.DMA,
                pltpu.SemaphoreType.DMA,
                pltpu.SemaphoreType.DMA,
                pltpu.SemaphoreType.DMA,
                pltpu.SemaphoreType.DMA,
                pltpu.SemaphoreType.DMA,
            ]
        ),
        compiler_params=pltpu.CompilerParams(use_tc_tiling_on_sc=False,
                                             needs_layout_passes=False),
    )
    def _sc_propagate(ylo_hbm, yhi_hbm, row_hbm, col_hbm, deg_hbm,
                      zlo_hbm, zhi_hbm, *rest):
        if prescale:
            (yslo_hbm, yshi_hbm, ridx0, ridx1,
             c00, c01, c02, c10, c11, c12,
             rows0, rows1, fa, fb, z_sh,
             semg0, semg1, semi0, semi1, sems0, sems1) = rest
            gsrc = (yslo_hbm, yshi_hbm)
        else:
            (ridx0, ridx1,
             c00, c01, c02, c10, c11, c12,
             rows0, rows1, fa, fb, z_sh,
             semg0, semg1, semi0, semi1, sems0, sems1) = rest
            gsrc = (ylo_hbm, yhi_hbm)
        c = lax.axis_index("c")
        s = lax.axis_index("s")

        zero16 = jnp.zeros((LANES,), jnp.float32)

        def _zb(i, _):
            for j in range(HALF // LANES):
                rows0[i, pl.ds(j * LANES, LANES)] = zero16
            return _

        lax.fori_loop(0, SUP_E, _zb, None)

        stripe0 = s * ROWS_PT

        def _zinit(k, _):
            pltpu.sync_copy(rows0, z_sh.at[pl.ds(stripe0 + k * SUP_E, SUP_E)])
            return _

        lax.fori_loop(0, ROWS_PT // SUP_E, _zinit, None)
        pltpu.sync_copy(rows0.at[pl.ds(0, ROWS_PT % SUP_E)],
                        z_sh.at[pl.ds(stripe0 + (ROWS_PT // SUP_E) * SUP_E,
                                      ROWS_PT % SUP_E)])

        def _scale_rows(buf, nrows, m):
            nv = (nrows + LANES - 1) // LANES
            for j in range(nv):
                sl = pl.ds(j * LANES, LANES)
                d = fa[sl] + fb[sl]
                fa[sl] = _newton_factor(d, m)

            def _mul_row(r, f16):
                for jj in range(HALF // LANES):
                    sl = pl.ds(jj * LANES, LANES)
                    buf[r, sl] = buf[r, sl] * f16

            def _sg(g, _):
                v = fa[pl.ds(g * LANES, LANES)]

                def _si(i, _2):
                    idx = jnp.full((LANES,), i, jnp.int32)
                    f16 = lax.gather(
                        v, idx[:, None],
                        lax.GatherDimensionNumbers(
                            offset_dims=(), collapsed_slice_dims=(0,),
                            start_index_map=(0,)),
                        slice_sizes=(1,),
                        mode=lax.GatherScatterMode.PROMISE_IN_BOUNDS)
                    _mul_row(g * LANES + i, f16)
                    return _2

                lax.fori_loop(0, LANES, _si, None)
                return _

            lax.fori_loop(0, nrows // LANES, _sg, None)

            def _sr(i, _):
                v = fa[pl.ds(i, LANES)]
                _mul_row(i, jnp.full((LANES,), v[0], jnp.float32))
                return _

            lax.fori_loop((nrows // LANES) * LANES, nrows, _sr, None)

        def _load_deg(r0, n):
            pltpu.sync_copy(deg_hbm.at[pl.ds(r0, n)], fa.at[pl.ds(0, n)])
            pltpu.sync_copy(deg_hbm.at[pl.ds(DEG_PAD + r0, n)],
                            fb.at[pl.ds(0, n)])

        if prescale:
            def _pre(x_ref, ydst):
                def _pc(k, _):
                    r0 = stripe0 + k * SUP_E
                    pltpu.sync_copy(x_ref.at[pl.ds(r0, SUP_E)], rows1)
                    _load_deg(r0, SUP_E)
                    _scale_rows(rows1, SUP_E, "rsqrt")
                    pltpu.sync_copy(rows1, ydst.at[pl.ds(r0, SUP_E)])
                    return _

                lax.fori_loop(0, ROWS_PT // SUP_E, _pc, None)

                def _ptail(tsz):
                    def f():
                        t0 = stripe0 + (ROWS_PT // SUP_E) * SUP_E
                        pltpu.sync_copy(x_ref.at[pl.ds(t0, tsz)],
                                        rows1.at[pl.ds(0, tsz)])
                        _load_deg(t0, tsz)
                        _scale_rows(rows1, tsz, "rsqrt")
                        pltpu.sync_copy(rows1.at[pl.ds(0, tsz)],
                                        ydst.at[pl.ds(t0, tsz)])
                    return f

                pl.when(s < NS - 1)(_ptail(ROWS_PT % SUP_E))
                pl.when(s == NS - 1)(_ptail(N_NODES - (NS - 1) * ROWS_PT
                                            - (ROWS_PT // SUP_E) * SUP_E))

            pl.when(c == 0)(lambda: _pre(ylo_hbm, yslo_hbm))
            pl.when(c == 1)(lambda: _pre(yhi_hbm, yshi_hbm))

        plsc.subcore_barrier()

        ridx = (ridx0, ridx1)
        cidx = ((c00, c01, c02), (c10, c11, c12))
        rows = (rows0, rows1)
        semg = (semg0, semg1)
        semi = (semi0, semi1)
        sems = (sems0, sems1)
        start_s = s * 390 + jnp.minimum(s, 10)

        def _run(y_ref):
            def fire(b, sup, drain_scatters):
                if drain_scatters:
                    pltpu.make_async_copy(y_ref.at[pl.ds(0, SUP_E)],
                                          rows[b], sems[b]).wait()
                e0 = (start_s + sup * SUPER) * CHUNK
                pltpu.async_copy(row_hbm.at[pl.ds(e0, SUP_E)], ridx[b],
                                 semi[b])
                for j in range(SUPER):
                    pltpu.async_copy(col_hbm.at[pl.ds(e0 + j * CHUNK, CHUNK)],
                                     cidx[b][j], semi[b])
                pltpu.make_async_copy(y_ref.at[pl.ds(0, SUP_E // 16)],
                                      rows[b].at[pl.ds(0, SUP_E // 16)],
                                      semi[b]).wait()
                for j in range(SUPER):
                    sl = pl.ds(j * CHUNK, CHUNK)
                    pltpu.async_copy(y_ref.at[ridx[b].at[sl]], rows[b].at[sl],
                                     semg[b])

            def process(b):
                pltpu.make_async_copy(y_ref.at[pl.ds(0, SUP_E)], rows[b],
                                      semg[b]).wait()
                for j in range(SUPER):
                    pltpu.async_copy(rows[b].at[pl.ds(j * CHUNK, CHUNK)],
                                     z_sh.at[cidx[b][j]], sems[b], add=True)

            fire(0, 0, False)
            fire(1, 1, False)
            process(0)

            def _body(m, _):
                fire(0, 2 * m + 2, True)
                process(1)
                fire(1, 2 * m + 3, True)
                process(0)
                return _

            lax.fori_loop(0, (SUPS_PT - 2) // 2, _body, None)
            process(1)
            pltpu.make_async_copy(y_ref.at[pl.ds(0, SUP_E)], rows[0],
                                  sems[0]).wait()
            pltpu.make_async_copy(y_ref.at[pl.ds(0, SUP_E)], rows[1],
                                  sems[1]).wait()

            def tail_one():
                e0 = (start_s + SUPS_PT * SUPER) * CHUNK
                sl = pl.ds(0, CHUNK)
                pltpu.sync_copy(row_hbm.at[pl.ds(e0, CHUNK)], ridx[0].at[sl])
                pltpu.sync_copy(col_hbm.at[pl.ds(e0, CHUNK)], cidx[0][0])
                pltpu.async_copy(y_ref.at[ridx[0].at[sl]], rows[0].at[sl],
                                 semg[0]).wait()
                pltpu.sync_copy(rows[0].at[sl], z_sh.at[cidx[0][0]],
                                add=True)

            pl.when(s < 10)(tail_one)

        pl.when(c == 0)(lambda: _run(gsrc[0]))
        pl.when(c == 1)(lambda: _run(gsrc[1]))
        plsc.subcore_barrier()

        def _cpout_to(dst):
            def _cp(k, _):
                r0 = stripe0 + k * SUP_E
                pltpu.sync_copy(z_sh.at[pl.ds(r0, SUP_E)], rows0)
                _load_deg(r0, SUP_E)
                _scale_rows(rows0, SUP_E, mode)
                pltpu.sync_copy(rows0, dst.at[pl.ds(r0, SUP_E)])
                return _

            lax.fori_loop(0, ROWS_PT // SUP_E, _cp, None)
            tail = ROWS_PT % SUP_E
            t0 = stripe0 + (ROWS_PT // SUP_E) * SUP_E
            pltpu.sync_copy(z_sh.at[pl.ds(t0, tail)], rows0.at[pl.ds(0, tail)])
            _load_deg(t0, tail)
            _scale_rows(rows0, tail, mode)
            pltpu.sync_copy(rows0.at[pl.ds(0, tail)], dst.at[pl.ds(t0, tail)])

        pl.when(c == 0)(lambda: _cpout_to(zlo_hbm))
        pl.when(c == 1)(lambda: _cpout_to(zhi_hbm))

    return _sc_propagate


_sc_propagate_l1 = _make_propagate("recip", True)
_sc_propagate_l2 = _make_propagate("rsqrt", False)


def _tc_dis_kernel(d0_ref, d1_ref, sqd_ref):
    deg = d0_ref[...] + d1_ref[...]
    sqd_ref[...] = jnp.where(deg > 0.0, jnp.sqrt(deg), 0.0)


def _tc_final_kernel(x_ref, sqd_ref, w1lo_ref, w1hi_ref, x2lo_ref, x2hi_ref,
                     out_ref):
    sq = sqd_ref[...]
    w1 = jnp.concatenate([w1lo_ref[...], w1hi_ref[...]], axis=1)
    x2 = jnp.concatenate([x2lo_ref[...], x2hi_ref[...]], axis=1)
    out_ref[...] = (x_ref[...] + sq * w1 + x2) * (1.0 / 3.0)


_R = 1000
_GRID = N_NODES // _R


def _row_spec(w):
    return pl.BlockSpec((_R, w), lambda i: (i, 0))


def kernel(user_weight, item_weight, edge_index):
    x0 = jnp.concatenate([user_weight, item_weight], axis=0)
    rowv = edge_index[0]
    colv = edge_index[1]

    xlo = jnp.concatenate([user_weight[:, :HALF], item_weight[:, :HALF]],
                          axis=0)
    xhi = jnp.concatenate([user_weight[:, HALF:], item_weight[:, HALF:]],
                          axis=0)

    degp = _sc_degree(colv)
    d0 = degp[:N_NODES].reshape(50, 1000)
    d1 = degp[DEG_PAD:DEG_PAD + N_NODES].reshape(50, 1000)

    sqd2 = pl.pallas_call(
        _tc_dis_kernel,
        out_shape=jax.ShapeDtypeStruct((50, 1000), jnp.float32),
    )(d0, d1)
    sqd = sqd2.reshape(N_NODES, 1)

    w1lo, w1hi, _, _ = _sc_propagate_l1(xlo, xhi, rowv, colv, degp)
    x2lo, x2hi = _sc_propagate_l2(w1lo, w1hi, rowv, colv, degp)

    out = pl.pallas_call(
        _tc_final_kernel,
        grid=(_GRID,),
        in_specs=[_row_spec(DIM), _row_spec(1), _row_spec(HALF),
                  _row_spec(HALF), _row_spec(HALF), _row_spec(HALF)],
        out_specs=_row_spec(DIM),
        out_shape=jax.ShapeDtypeStruct((N_NODES, DIM), jnp.float32),
    )(x0, sqd, w1lo, w1hi, x2lo, x2hi)

    return out[:N_USERS], out[N_USERS:]
